# baseline (speedup 1.0000x reference)
import functools

import jax
import jax.numpy as jnp
from jax import lax
from jax.experimental import pallas as pl
from jax.experimental.pallas import tpu as pltpu
from jax.experimental.pallas import tpu_sc as plsc

N = 10000
E = 320000
D_IN = 128
D_HID = 256
D_OUT = 128

NC = 2
NS = 16
K = 128

E2 = E + N
EPT = -(-E2 // (NC * NS * K)) * K
CH = EPT // K
EPAD = NC * NS * EPT
TRASH = N
NACC = 10112
RPT = NACC // NS
DEGW = 16

_mesh = plsc.VectorSubcoreMesh(core_axis_name="c", subcore_axis_name="s")


@functools.partial(
    pl.kernel,
    out_type=jax.ShapeDtypeStruct((NC, NACC, DEGW), jnp.float32),
    mesh=_mesh,
    scratch_types=[
        pltpu.VMEM((CH, K), jnp.int32),
        pltpu.VMEM((K, DEGW), jnp.float32),
        pltpu.VMEM_SHARED((NACC, DEGW), jnp.float32),
        pltpu.SemaphoreType.DMA,
        pltpu.SemaphoreType.DMA,
        pltpu.SemaphoreType.DMA,
    ],
)
def _sc_deg(cols_hbm, ones_hbm, zeros_hbm, out_hbm, cidx, ones_v, acc,
            d0, d1, d2):
    c = lax.axis_index("c")
    s = lax.axis_index("s")
    dsem = (d0, d1, d2)
    pltpu.sync_copy(zeros_hbm.at[pl.ds(s * RPT, RPT)], acc.at[pl.ds(s * RPT, RPT)])
    pltpu.sync_copy(cols_hbm.at[c, s], cidx)
    pltpu.sync_copy(ones_hbm, ones_v)
    plsc.subcore_barrier()

    def dstart(b, j):
        pltpu.async_copy(ones_v, acc.at[cidx.at[j]], dsem[b], add=True)

    def dwait(b, j):
        pltpu.make_async_copy(ones_v, acc.at[cidx.at[j]], dsem[b]).wait()

    for b in range(3):
        dstart(b, b)

    def group(g, carry):
        j0 = 3 * g + 3
        for b in range(3):
            dwait(b, j0 + b - 3)
            dstart(b, j0 + b)
        return carry

    lax.fori_loop(0, CH // 3 - 1, group, 0)
    for b in range(3):
        dwait(b, CH - 3 + b)
    plsc.subcore_barrier()
    pltpu.sync_copy(acc.at[pl.ds(s * RPT, RPT)], out_hbm.at[c, pl.ds(s * RPT, RPT)])


@functools.partial(
    pl.kernel,
    out_type=jax.ShapeDtypeStruct((NC, NACC, D_IN), jnp.float32),
    mesh=_mesh,
    scratch_types=[
        pltpu.VMEM((3, 2, K), jnp.int32),
        pltpu.VMEM((3, K, D_IN), jnp.float32),
        pltpu.VMEM_SHARED((NACC, D_IN), jnp.float32),
        pltpu.SemaphoreType.DMA,
        pltpu.SemaphoreType.DMA,
        pltpu.SemaphoreType.DMA,
        pltpu.SemaphoreType.DMA,
        pltpu.SemaphoreType.DMA,
        pltpu.SemaphoreType.DMA,
    ],
)
def _sc_msg(xs_hbm, idx_hbm, zeros_hbm, out_hbm,
            idx, msg, acc, g0, g1, g2, s0, s1, s2):
    c = lax.axis_index("c")
    s = lax.axis_index("s")
    gsem = (g0, g1, g2)
    ssem = (s0, s1, s2)
    pltpu.sync_copy(zeros_hbm.at[pl.ds(s * RPT, RPT)], acc.at[pl.ds(s * RPT, RPT)])

    def iload(b, j):
        pltpu.sync_copy(idx_hbm.at[c, s, j], idx.at[b])

    def gstart(b):
        pltpu.async_copy(xs_hbm.at[idx.at[b, 0]], msg.at[b], gsem[b])

    def gwait(b):
        pltpu.make_async_copy(xs_hbm.at[idx.at[b, 0]], msg.at[b], gsem[b]).wait()

    def sstart(b):
        pltpu.async_copy(msg.at[b], acc.at[idx.at[b, 1]], ssem[b], add=True)

    def swait(b):
        pltpu.make_async_copy(msg.at[b], acc.at[idx.at[b, 1]], ssem[b]).wait()

    iload(0, 0)
    iload(1, 1)
    iload(2, 2)
    plsc.subcore_barrier()

    gstart(0)
    gstart(1)
    gwait(0)
    sstart(0)
    gstart(2)

    def group(g, carry):
        j0 = 3 * g + 1
        for bb in range(3):
            j = j0 + bb
            b = (1 + bb) % 3
            pb = bb
            gwait(b)
            sstart(b)
            swait(pb)
            iload(pb, j + 2)
            gstart(pb)
        return carry

    lax.fori_loop(0, (CH - 3) // 3, group, 0)
    gwait(1)
    sstart(1)
    gwait(2)
    sstart(2)
    swait(0)
    swait(1)
    swait(2)
    plsc.subcore_barrier()
    pltpu.sync_copy(acc.at[pl.ds(s * RPT, RPT)], out_hbm.at[c, pl.ds(s * RPT, RPT)])


BLK = 1000
GRID = N // BLK


def _tc_scale_in(deg_ref, x_ref, xs_ref, dinv_ref):
    d = deg_ref[0, :, 0:1] + deg_ref[1, :, 0:1]
    dinv = lax.rsqrt(d)
    dinv_ref[...] = jnp.broadcast_to(dinv, (BLK, 16))
    xs_ref[...] = x_ref[...] * jnp.broadcast_to(dinv, (BLK, D_IN))


def _tc_mid(t1_ref, dinv_ref, w1_ref, b1_ref, w2_ref, xs2_ref):
    dinvb = jnp.broadcast_to(dinv_ref[:, 0:1], (BLK, D_IN))
    t1 = (t1_ref[0] + t1_ref[1]) * dinvb
    h1 = lax.dot_general(t1, w1_ref[...], (((1,), (1,)), ((), ())),
                         preferred_element_type=jnp.float32) + b1_ref[...]
    y = jnp.maximum(h1, 0.0)
    h2 = lax.dot_general(y, w2_ref[...], (((1,), (1,)), ((), ())),
                         preferred_element_type=jnp.float32)
    xs2_ref[...] = h2 * dinvb


def _tc_out(t2_ref, dinv_ref, b2_ref, out_ref):
    dinvb = jnp.broadcast_to(dinv_ref[:, 0:1], (BLK, D_OUT))
    out_ref[...] = (t2_ref[0] + t2_ref[1]) * dinvb + b2_ref[...]


def _acc_spec(width):
    return pl.BlockSpec((NC, BLK, width), lambda i: (0, i, 0))


def _row_spec(width):
    return pl.BlockSpec((BLK, width), lambda i: (i, 0))


def _full_spec(shape):
    return pl.BlockSpec(shape, lambda i: tuple(0 for _ in shape))


def kernel(x, edge_index, W1, b1, W2, b2):
    f32 = jnp.float32
    loop = jnp.arange(N, dtype=jnp.int32)
    pad = EPAD - E2
    prange = jnp.arange(pad, dtype=jnp.int32)
    rows = jnp.concatenate([edge_index[0], loop,
                            prange % N]).reshape(NC, NS, CH, K)
    cols = jnp.concatenate([edge_index[1], loop,
                            TRASH + prange % (NACC - N)]).reshape(NC, NS, CH, K)
    idx = jnp.stack([rows, cols], axis=3)

    ones_deg = jnp.ones((K, DEGW), f32)
    zeros_deg = jnp.zeros((NACC, DEGW), f32)
    zeros_msg = jnp.zeros((NACC, D_IN), f32)

    degp = _sc_deg(cols, ones_deg, zeros_deg)

    xs1, dinv = pl.pallas_call(
        _tc_scale_in,
        grid=(GRID,),
        in_specs=[_acc_spec(DEGW), _row_spec(D_IN)],
        out_specs=[_row_spec(D_IN), _row_spec(16)],
        out_shape=[jax.ShapeDtypeStruct((N, D_IN), f32),
                   jax.ShapeDtypeStruct((N, 16), f32)],
    )(degp, x)

    t1p = _sc_msg(xs1, idx, zeros_msg)

    xs2 = pl.pallas_call(
        _tc_mid,
        grid=(GRID,),
        in_specs=[_acc_spec(D_IN), _row_spec(16),
                  _full_spec((D_HID, D_IN)), _full_spec((1, D_HID)),
                  _full_spec((D_OUT, D_HID))],
        out_specs=_row_spec(D_IN),
        out_shape=jax.ShapeDtypeStruct((N, D_IN), f32),
    )(t1p, dinv, W1, b1.reshape(1, D_HID), W2)

    t2p = _sc_msg(xs2, idx, zeros_msg)

    out = pl.pallas_call(
        _tc_out,
        grid=(GRID,),
        in_specs=[_acc_spec(D_IN), _row_spec(16), _full_spec((1, D_OUT))],
        out_specs=_row_spec(D_OUT),
        out_shape=jax.ShapeDtypeStruct((N, D_OUT), f32),
    )(t2p, dinv, b2.reshape(1, D_OUT))

    return out


# speedup vs baseline: 1.0502x; 1.0502x over previous
import functools

import jax
import jax.numpy as jnp
from jax import lax
from jax.experimental import pallas as pl
from jax.experimental.pallas import tpu as pltpu
from jax.experimental.pallas import tpu_sc as plsc

N = 10000
E = 320000
D_IN = 128
D_HID = 256
D_OUT = 128

NC = 2
NS = 16
K = 128

E2 = E + N
EPT = -(-E2 // (NC * NS * K)) * K
CH = EPT // K
EPAD = NC * NS * EPT
TRASH = N
NACC = 10112
RPT = NACC // NS
DEGW = 16

_mesh = plsc.VectorSubcoreMesh(core_axis_name="c", subcore_axis_name="s")


@functools.partial(
    pl.kernel,
    out_type=jax.ShapeDtypeStruct((NC, NACC, DEGW), jnp.float32),
    mesh=_mesh,
    scratch_types=[
        pltpu.VMEM((CH, K), jnp.int32),
        pltpu.VMEM((K, DEGW), jnp.float32),
        pltpu.VMEM((K, DEGW), jnp.float32),
        pltpu.VMEM_SHARED((NACC, DEGW), jnp.float32),
        pltpu.SemaphoreType.DMA,
        pltpu.SemaphoreType.DMA,
        pltpu.SemaphoreType.DMA,
    ],
)
def _sc_deg(cols_hbm, out_hbm, cidx, ones_v, zbuf, acc, d0, d1, d2):
    c = lax.axis_index("c")
    s = lax.axis_index("s")
    dsem = (d0, d1, d2)
    pltpu.sync_copy(cols_hbm.at[c, s], cidx)

    def fill(r, carry):
        ones_v[r, :] = jnp.ones((DEGW,), jnp.float32)
        zbuf[r, :] = jnp.zeros((DEGW,), jnp.float32)
        return carry

    lax.fori_loop(0, K, fill, 0)
    for m in range(RPT // K):
        pltpu.sync_copy(zbuf, acc.at[pl.ds(s * RPT + m * K, K)])
    pltpu.sync_copy(zbuf.at[pl.ds(0, RPT % K)],
                    acc.at[pl.ds(s * RPT + (RPT // K) * K, RPT % K)])
    plsc.subcore_barrier()

    def dstart(b, j):
        pltpu.async_copy(ones_v, acc.at[cidx.at[j]], dsem[b], add=True)

    def dwait(b, j):
        pltpu.make_async_copy(ones_v, acc.at[cidx.at[j]], dsem[b]).wait()

    for b in range(3):
        dstart(b, b)

    def group(g, carry):
        j0 = 3 * g + 3
        for b in range(3):
            dwait(b, j0 + b - 3)
            dstart(b, j0 + b)
        return carry

    lax.fori_loop(0, CH // 3 - 1, group, 0)
    for b in range(3):
        dwait(b, CH - 3 + b)
    plsc.subcore_barrier()
    pltpu.sync_copy(acc.at[pl.ds(s * RPT, RPT)], out_hbm.at[c, pl.ds(s * RPT, RPT)])


@functools.partial(
    pl.kernel,
    out_type=jax.ShapeDtypeStruct((NC, NACC, D_IN), jnp.float32),
    mesh=_mesh,
    scratch_types=[
        pltpu.VMEM((3, 2, K), jnp.int32),
        pltpu.VMEM((3, K, D_IN), jnp.float32),
        pltpu.VMEM_SHARED((NACC, D_IN), jnp.float32),
        pltpu.SemaphoreType.DMA,
        pltpu.SemaphoreType.DMA,
        pltpu.SemaphoreType.DMA,
        pltpu.SemaphoreType.DMA,
        pltpu.SemaphoreType.DMA,
        pltpu.SemaphoreType.DMA,
    ],
)
def _sc_msg(xs_hbm, idx_hbm, out_hbm,
            idx, msg, acc, g0, g1, g2, s0, s1, s2):
    c = lax.axis_index("c")
    s = lax.axis_index("s")
    gsem = (g0, g1, g2)
    ssem = (s0, s1, s2)

    def zrow(r, carry):
        for k in range(D_IN // 16):
            msg[0, r, pl.ds(16 * k, 16)] = jnp.zeros((16,), jnp.float32)
        return carry

    lax.fori_loop(0, K, zrow, 0)
    for m in range(RPT // K):
        pltpu.sync_copy(msg.at[0], acc.at[pl.ds(s * RPT + m * K, K)])
    pltpu.sync_copy(msg.at[0, pl.ds(0, RPT % K)],
                    acc.at[pl.ds(s * RPT + (RPT // K) * K, RPT % K)])

    def iload(b, j):
        pltpu.sync_copy(idx_hbm.at[c, s, j], idx.at[b])

    def gstart(b):
        pltpu.async_copy(xs_hbm.at[idx.at[b, 0]], msg.at[b], gsem[b])

    def gwait(b):
        pltpu.make_async_copy(xs_hbm.at[idx.at[b, 0]], msg.at[b], gsem[b]).wait()

    def sstart(b):
        pltpu.async_copy(msg.at[b], acc.at[idx.at[b, 1]], ssem[b], add=True)

    def swait(b):
        pltpu.make_async_copy(msg.at[b], acc.at[idx.at[b, 1]], ssem[b]).wait()

    iload(0, 0)
    iload(1, 1)
    iload(2, 2)
    plsc.subcore_barrier()

    gstart(0)
    gstart(1)
    gwait(0)
    sstart(0)
    gstart(2)

    def group(g, carry):
        j0 = 3 * g + 1
        for bb in range(3):
            j = j0 + bb
            b = (1 + bb) % 3
            pb = bb
            gwait(b)
            sstart(b)
            swait(pb)
            iload(pb, j + 2)
            gstart(pb)
        return carry

    lax.fori_loop(0, (CH - 3) // 3, group, 0)
    gwait(1)
    sstart(1)
    gwait(2)
    sstart(2)
    swait(0)
    swait(1)
    swait(2)
    plsc.subcore_barrier()
    pltpu.sync_copy(acc.at[pl.ds(s * RPT, RPT)], out_hbm.at[c, pl.ds(s * RPT, RPT)])


BLK = 1000
GRID = N // BLK


def _tc_scale_in(deg_ref, x_ref, xs_ref, dinv_ref):
    d = deg_ref[0, :, 0:1] + deg_ref[1, :, 0:1]
    dinv = lax.rsqrt(d)
    dinv_ref[...] = jnp.broadcast_to(dinv, (BLK, 16))
    xs_ref[...] = x_ref[...] * jnp.broadcast_to(dinv, (BLK, D_IN))


def _tc_mid(t1_ref, dinv_ref, w1_ref, b1_ref, w2_ref, xs2_ref):
    dinvb = jnp.broadcast_to(dinv_ref[:, 0:1], (BLK, D_IN))
    t1 = (t1_ref[0] + t1_ref[1]) * dinvb
    h1 = lax.dot_general(t1, w1_ref[...], (((1,), (1,)), ((), ())),
                         preferred_element_type=jnp.float32) + b1_ref[...]
    y = jnp.maximum(h1, 0.0)
    h2 = lax.dot_general(y, w2_ref[...], (((1,), (1,)), ((), ())),
                         preferred_element_type=jnp.float32)
    xs2_ref[...] = h2 * dinvb


def _tc_out(t2_ref, dinv_ref, b2_ref, out_ref):
    dinvb = jnp.broadcast_to(dinv_ref[:, 0:1], (BLK, D_OUT))
    out_ref[...] = (t2_ref[0] + t2_ref[1]) * dinvb + b2_ref[...]


def _acc_spec(width):
    return pl.BlockSpec((NC, BLK, width), lambda i: (0, i, 0))


def _row_spec(width):
    return pl.BlockSpec((BLK, width), lambda i: (i, 0))


def _full_spec(shape):
    return pl.BlockSpec(shape, lambda i: tuple(0 for _ in shape))


def kernel(x, edge_index, W1, b1, W2, b2):
    f32 = jnp.float32
    loop = jnp.arange(N, dtype=jnp.int32)
    pad = EPAD - E2
    prange = jnp.arange(pad, dtype=jnp.int32)
    rows = jnp.concatenate([edge_index[0], loop,
                            prange % N]).reshape(NC, NS, CH, K)
    cols = jnp.concatenate([edge_index[1], loop,
                            TRASH + prange % (NACC - N)]).reshape(NC, NS, CH, K)
    idx = jnp.stack([rows, cols], axis=3)

    degp = _sc_deg(cols)

    xs1, dinv = pl.pallas_call(
        _tc_scale_in,
        grid=(GRID,),
        in_specs=[_acc_spec(DEGW), _row_spec(D_IN)],
        out_specs=[_row_spec(D_IN), _row_spec(16)],
        out_shape=[jax.ShapeDtypeStruct((N, D_IN), f32),
                   jax.ShapeDtypeStruct((N, 16), f32)],
    )(degp, x)

    t1p = _sc_msg(xs1, idx)

    xs2 = pl.pallas_call(
        _tc_mid,
        grid=(GRID,),
        in_specs=[_acc_spec(D_IN), _row_spec(16),
                  _full_spec((D_HID, D_IN)), _full_spec((1, D_HID)),
                  _full_spec((D_OUT, D_HID))],
        out_specs=_row_spec(D_IN),
        out_shape=jax.ShapeDtypeStruct((N, D_IN), f32),
    )(t1p, dinv, W1, b1.reshape(1, D_HID), W2)

    t2p = _sc_msg(xs2, idx)

    out = pl.pallas_call(
        _tc_out,
        grid=(GRID,),
        in_specs=[_acc_spec(D_IN), _row_spec(16), _full_spec((1, D_OUT))],
        out_specs=_row_spec(D_OUT),
        out_shape=jax.ShapeDtypeStruct((N, D_OUT), f32),
    )(t2p, dinv, b2.reshape(1, D_OUT))

    return out
